# Initial kernel scaffold; baseline (speedup 1.0000x reference)
import jax
import jax.numpy as jnp
from jax.experimental import pallas as pl


def kernel(x, edge_index, edge_weight, W1, b1, W2, b2):
    raise NotImplementedError("write your pallas kernel here")


# speedup vs baseline: 8.1552x; 8.1552x over previous
import functools

import jax
import jax.numpy as jnp
from jax import lax
from jax.experimental import pallas as pl
from jax.experimental.pallas import tpu as pltpu
from jax.experimental.pallas import tpu_sc as plsc

N = 10000
D = 128
E = 320000

NC = 2
NS = 16
NW = NC * NS
L = 16

NPAD = 10240
CH = 80
EW_PER_W = E // NW
MSG_CHUNKS = EW_PER_W // CH
E_PER_T = E // NS
DEG_CHUNKS = E_PER_T // CH
ZR = 128
ROWS_PER_T = NPAD // NS
DIS_PER_T = NPAD // NW


def _make_deg_kernel():
    mesh = plsc.VectorSubcoreMesh(
        core_axis_name="c", subcore_axis_name="s", num_cores=NC,
        num_subcores=NS)

    @functools.partial(
        pl.kernel,
        out_type=jax.ShapeDtypeStruct((NPAD,), jnp.float32),
        mesh=mesh,
        compiler_params=pltpu.CompilerParams(needs_layout_passes=False),
        scratch_types=dict(
            dstv=pltpu.VMEM((CH,), jnp.int32),
            ewv=pltpu.VMEM((CH,), jnp.float32),
            degv=pltpu.VMEM((DIS_PER_T,), jnp.float32),
            onev=pltpu.VMEM((NPAD // NS,), jnp.float32),
            deg_sh=pltpu.VMEM_SHARED((NPAD,), jnp.float32),
        ),
    )
    def deg_kernel(dst_hbm, ew_hbm, deg_hbm, dstv, ewv, degv, onev, deg_sh):
        cid = lax.axis_index("c")
        sid = lax.axis_index("s")

        def init_one(i, _):
            onev[pl.ds(i * L, L)] = jnp.full((L,), 1.0, jnp.float32)
            return 0
        lax.fori_loop(0, (NPAD // NS) // L, init_one, 0)
        pltpu.sync_copy(onev, deg_sh.at[pl.ds(sid * (NPAD // NS), NPAD // NS)])
        plsc.subcore_barrier()

        def scat(k, _):
            base = sid * E_PER_T + k * CH
            pltpu.sync_copy(dst_hbm.at[pl.ds(base, CH)], dstv)
            pltpu.sync_copy(ew_hbm.at[pl.ds(base, CH)], ewv)
            pltpu.sync_copy(ewv, deg_sh.at[dstv], add=True)
            return 0
        lax.fori_loop(0, DEG_CHUNKS, scat, 0)
        plsc.subcore_barrier()

        off = (cid * NS + sid) * DIS_PER_T
        pltpu.sync_copy(deg_sh.at[pl.ds(off, DIS_PER_T)], degv)
        pltpu.sync_copy(degv, deg_hbm.at[pl.ds(off, DIS_PER_T)])

    return deg_kernel


def _make_msg_kernel():
    mesh = plsc.VectorSubcoreMesh(
        core_axis_name="c", subcore_axis_name="s", num_cores=NC,
        num_subcores=NS)

    @functools.partial(
        pl.kernel,
        out_type=(jax.ShapeDtypeStruct((NPAD, D), jnp.float32),
                  jax.ShapeDtypeStruct((NPAD, D), jnp.float32)),
        mesh=mesh,
        compiler_params=pltpu.CompilerParams(needs_layout_passes=False),
        scratch_types=dict(
            srcv=pltpu.VMEM((CH,), jnp.int32),
            dstv=pltpu.VMEM((CH,), jnp.int32),
            ewv=pltpu.VMEM((CH,), jnp.float32),
            rows=pltpu.VMEM((CH, D), jnp.float32),
            zbuf=pltpu.VMEM((ZR, D), jnp.float32),
            sem=pltpu.SemaphoreType.DMA,
            acc=pltpu.VMEM_SHARED((NPAD, D), jnp.float32),
        ),
    )
    def msg_kernel(src_hbm, dst_hbm, ew_hbm, y_hbm, p0_hbm, p1_hbm,
                   srcv, dstv, ewv, rows, zbuf, sem, acc):
        cid = lax.axis_index("c")
        sid = lax.axis_index("s")
        w = cid * NS + sid

        def zrow(r, _):
            for j in range(D // L):
                zbuf[r, pl.ds(j * L, L)] = jnp.zeros((L,), jnp.float32)
            return 0
        lax.fori_loop(0, ZR, zrow, 0)
        for t in range(ROWS_PER_T // ZR):
            pltpu.sync_copy(zbuf, acc.at[pl.ds(sid * ROWS_PER_T + t * ZR, ZR)])
        plsc.subcore_barrier()

        def chunk(k, _):
            base = w * EW_PER_W + k * CH
            pltpu.sync_copy(src_hbm.at[pl.ds(base, CH)], srcv)
            pltpu.sync_copy(dst_hbm.at[pl.ds(base, CH)], dstv)
            pltpu.sync_copy(ew_hbm.at[pl.ds(base, CH)], ewv)
            pltpu.async_copy(y_hbm.at[srcv], rows, sem).wait()

            def scale(e, _):
                s = plsc.load_gather(ewv, [jnp.full((L,), e, jnp.int32)])
                for j in range(D // L):
                    rows[e, pl.ds(j * L, L)] = rows[e, pl.ds(j * L, L)] * s
                return 0
            lax.fori_loop(0, CH, scale, 0)
            pltpu.sync_copy(rows, acc.at[dstv], add=True)
            return 0
        lax.fori_loop(0, MSG_CHUNKS, chunk, 0)
        plsc.subcore_barrier()

        ro = sid * ROWS_PER_T

        @pl.when(cid == 0)
        def _():
            for t in range(ROWS_PER_T // ZR):
                pltpu.sync_copy(acc.at[pl.ds(ro + t * ZR, ZR)], zbuf)
                pltpu.sync_copy(zbuf, p0_hbm.at[pl.ds(ro + t * ZR, ZR)])

        @pl.when(cid == 1)
        def _():
            for t in range(ROWS_PER_T // ZR):
                pltpu.sync_copy(acc.at[pl.ds(ro + t * ZR, ZR)], zbuf)
                pltpu.sync_copy(zbuf, p1_hbm.at[pl.ds(ro + t * ZR, ZR)])

    return msg_kernel


BR = 512
NB = NPAD // BR

_row_spec = pl.BlockSpec((BR, D), lambda i: (i, 0))
_col_spec = pl.BlockSpec((BR, 1), lambda i: (i, 0))
_mat_spec = pl.BlockSpec((D, D), lambda i: (0, 0))
_vec_spec = pl.BlockSpec((1, D), lambda i: (0, 0))

_HI = lax.Precision.HIGHEST


def _k1_body(x_ref, w1_ref, deg_ref, xw_ref, y_ref, dis_ref):
    dis = lax.rsqrt(deg_ref[...])
    dis_ref[...] = dis
    xw = lax.dot_general(x_ref[...], w1_ref[...], (((1,), (0,)), ((), ())),
                         precision=_HI)
    xw_ref[...] = xw
    y_ref[...] = dis * xw


def _k3_body(p0_ref, p1_ref, xw1_ref, dis_ref, b1_ref, w2_ref,
             xw2_ref, y2_ref):
    dis = dis_ref[...]
    h = dis * (p0_ref[...] + p1_ref[...]) + dis * dis * xw1_ref[...]
    h = jnp.maximum(h + b1_ref[...], 0.0)
    xw2 = lax.dot_general(h, w2_ref[...], (((1,), (0,)), ((), ())),
                          precision=_HI)
    xw2_ref[...] = xw2
    y2_ref[...] = dis * xw2


def _k5_body(q0_ref, q1_ref, xw2_ref, dis_ref, b2_ref, out_ref):
    dis = dis_ref[...]
    out_ref[...] = (dis * (q0_ref[...] + q1_ref[...])
                    + dis * dis * xw2_ref[...] + b2_ref[...])


_deg_kernel = _make_deg_kernel()
_msg_kernel = _make_msg_kernel()


@jax.jit
def kernel(x, edge_index, edge_weight, W1, b1, W2, b2):
    src = edge_index[0]
    dst = edge_index[1]
    xp = jnp.pad(x, ((0, NPAD - N), (0, 0)))

    deg = _deg_kernel(dst, edge_weight).reshape(NPAD, 1)

    xw1, y1, dis = pl.pallas_call(
        _k1_body,
        grid=(NB,),
        in_specs=[_row_spec, _mat_spec, _col_spec],
        out_specs=[_row_spec, _row_spec, _col_spec],
        out_shape=[jax.ShapeDtypeStruct((NPAD, D), jnp.float32),
                   jax.ShapeDtypeStruct((NPAD, D), jnp.float32),
                   jax.ShapeDtypeStruct((NPAD, 1), jnp.float32)],
    )(xp, W1, deg)

    p0, p1 = _msg_kernel(src, dst, edge_weight, y1)

    xw2, y2 = pl.pallas_call(
        _k3_body,
        grid=(NB,),
        in_specs=[_row_spec, _row_spec, _row_spec, _col_spec, _vec_spec,
                  _mat_spec],
        out_specs=[_row_spec, _row_spec],
        out_shape=[jax.ShapeDtypeStruct((NPAD, D), jnp.float32)] * 2,
    )(p0, p1, xw1, dis, b1.reshape(1, D), W2)

    q0, q1 = _msg_kernel(src, dst, edge_weight, y2)

    out = pl.pallas_call(
        _k5_body,
        grid=(NB,),
        in_specs=[_row_spec, _row_spec, _row_spec, _col_spec, _vec_spec],
        out_specs=_row_spec,
        out_shape=jax.ShapeDtypeStruct((NPAD, D), jnp.float32),
    )(q0, q1, xw2, dis, b2.reshape(1, D))
    return out[:N]


# speedup vs baseline: 9.4179x; 1.1548x over previous
import functools

import jax
import jax.numpy as jnp
from jax import lax
from jax.experimental import pallas as pl
from jax.experimental.pallas import tpu as pltpu
from jax.experimental.pallas import tpu_sc as plsc

N = 10000
D = 128
E = 320000

NC = 2
NS = 16
NW = NC * NS
L = 16

NPAD = 10240
CH = 128
MSG_CHUNKS = 80
EW_PER_W = MSG_CHUNKS * CH
E_PAD = NW * EW_PER_W
ZR = 128
IG = 16
ROWS_PER_T = NPAD // NS
DIS_PER_T = NPAD // NW


def _make_deg_kernel():
    mesh = plsc.VectorSubcoreMesh(
        core_axis_name="c", subcore_axis_name="s", num_cores=NC,
        num_subcores=NS)

    @functools.partial(
        pl.kernel,
        out_type=(jax.ShapeDtypeStruct((NPAD,), jnp.float32),
                  jax.ShapeDtypeStruct((NPAD,), jnp.float32)),
        mesh=mesh,
        compiler_params=pltpu.CompilerParams(needs_layout_passes=False),
        scratch_types=dict(
            dstall=pltpu.VMEM((MSG_CHUNKS, CH), jnp.int32),
            ewall=pltpu.VMEM((MSG_CHUNKS, CH), jnp.float32),
            degv=pltpu.VMEM((NPAD // NS,), jnp.float32),
            onev=pltpu.VMEM((NPAD // NS,), jnp.float32),
            sem=pltpu.SemaphoreType.DMA,
            deg_sh=pltpu.VMEM_SHARED((NPAD,), jnp.float32),
        ),
    )
    def deg_kernel(dst3_hbm, ew3_hbm, d0_hbm, d1_hbm, dstall, ewall, degv,
                   onev, sem, deg_sh):
        cid = lax.axis_index("c")
        sid = lax.axis_index("s")
        w = cid * NS + sid

        fill = jnp.where(cid == 0, 1.0, 0.0).astype(jnp.float32)

        def init_one(i, _):
            onev[pl.ds(i * L, L)] = jnp.broadcast_to(fill, (L,))
            return 0
        lax.fori_loop(0, (NPAD // NS) // L, init_one, 0)
        pltpu.sync_copy(onev, deg_sh.at[pl.ds(sid * (NPAD // NS), NPAD // NS)])

        pltpu.sync_copy(dst3_hbm.at[w], dstall)
        pltpu.sync_copy(ew3_hbm.at[w], ewall)
        plsc.subcore_barrier()

        def fire(k, _):
            pltpu.make_async_copy(
                ewall.at[k], deg_sh.at[dstall.at[k]], sem).start(add=True)
            return 0
        lax.fori_loop(0, MSG_CHUNKS, fire, 0)

        def drain(k, _):
            pltpu.make_async_copy(
                ewall.at[0], deg_sh.at[dstall.at[0]], sem).wait()
            return 0
        lax.fori_loop(0, MSG_CHUNKS, drain, 0)
        plsc.subcore_barrier()

        pt = NPAD // NS
        off = sid * pt
        pltpu.sync_copy(deg_sh.at[pl.ds(off, pt)], degv)

        @pl.when(cid == 0)
        def _():
            pltpu.sync_copy(degv, d0_hbm.at[pl.ds(off, pt)])

        @pl.when(cid == 1)
        def _():
            pltpu.sync_copy(degv, d1_hbm.at[pl.ds(off, pt)])

    return deg_kernel


def _make_msg_kernel():
    mesh = plsc.VectorSubcoreMesh(
        core_axis_name="c", subcore_axis_name="s", num_cores=NC,
        num_subcores=NS)

    @functools.partial(
        pl.kernel,
        out_type=(jax.ShapeDtypeStruct((NPAD, D), jnp.float32),
                  jax.ShapeDtypeStruct((NPAD, D), jnp.float32)),
        mesh=mesh,
        compiler_params=pltpu.CompilerParams(needs_layout_passes=False),
        scratch_types=dict(
            srcall=pltpu.VMEM((MSG_CHUNKS, CH), jnp.int32),
            dstg=pltpu.VMEM((IG, CH), jnp.int32),
            ewg=pltpu.VMEM((IG, CH), jnp.float32),
            rows0=pltpu.VMEM((CH, D), jnp.float32),
            rows1=pltpu.VMEM((CH, D), jnp.float32),
            sem0=pltpu.SemaphoreType.DMA,
            sem1=pltpu.SemaphoreType.DMA,
            acc=pltpu.VMEM_SHARED((NPAD, D), jnp.float32),
        ),
    )
    def msg_kernel(src3_hbm, dst3_hbm, ew3_hbm, y_hbm, p0_hbm, p1_hbm,
                   srcall, dstg, ewg, rows0, rows1, sem0, sem1, acc):
        cid = lax.axis_index("c")
        sid = lax.axis_index("s")
        w = cid * NS + sid
        bufs = ((rows0, sem0), (rows1, sem1))

        def zrow(r, _):
            for j in range(D // L):
                rows0[r, pl.ds(j * L, L)] = jnp.zeros((L,), jnp.float32)
            return 0
        lax.fori_loop(0, CH, zrow, 0)
        pltpu.sync_copy(src3_hbm.at[w], srcall)
        for t in range(ROWS_PER_T // ZR):
            pltpu.sync_copy(rows0, acc.at[pl.ds(sid * ROWS_PER_T + t * ZR, ZR)])
        plsc.subcore_barrier()

        for b, (rows, sem) in enumerate(bufs):
            pltpu.make_async_copy(y_hbm.at[srcall.at[b]], rows, sem).start()

        for g in range(MSG_CHUNKS // IG):
            pltpu.sync_copy(dst3_hbm.at[w, pl.ds(g * IG, IG)], dstg)
            pltpu.sync_copy(ew3_hbm.at[w, pl.ds(g * IG, IG)], ewg)
            for t in range(IG // 2):
                for b, (rows, sem) in enumerate(bufs):
                    kg = 2 * t + b
                    k = g * IG + kg
                    pltpu.make_async_copy(y_hbm.at[srcall.at[k]], rows,
                                          sem).wait()

                    def scale(e, _, kg=kg, rows=rows):
                        ew = plsc.load_gather(
                            ewg, [jnp.full((L,), kg, jnp.int32),
                                  jnp.full((L,), e, jnp.int32)])
                        for j in range(D // L):
                            rows[e, pl.ds(j * L, L)] = (
                                rows[e, pl.ds(j * L, L)] * ew)
                        return 0
                    lax.fori_loop(0, CH, scale, 0)
                    pltpu.sync_copy(rows, acc.at[dstg.at[kg]], add=True)
                    if k + 2 < MSG_CHUNKS:
                        pltpu.make_async_copy(y_hbm.at[srcall.at[k + 2]],
                                              rows, sem).start()
        plsc.subcore_barrier()

        ro = sid * ROWS_PER_T

        @pl.when(cid == 0)
        def _():
            for t in range(ROWS_PER_T // ZR):
                pltpu.sync_copy(acc.at[pl.ds(ro + t * ZR, ZR)], rows0)
                pltpu.sync_copy(rows0, p0_hbm.at[pl.ds(ro + t * ZR, ZR)])

        @pl.when(cid == 1)
        def _():
            for t in range(ROWS_PER_T // ZR):
                pltpu.sync_copy(acc.at[pl.ds(ro + t * ZR, ZR)], rows0)
                pltpu.sync_copy(rows0, p1_hbm.at[pl.ds(ro + t * ZR, ZR)])

    return msg_kernel


BR = 512
NB = NPAD // BR

_row_spec = pl.BlockSpec((BR, D), lambda i: (i, 0))
_col_spec = pl.BlockSpec((BR, 1), lambda i: (i, 0))
_mat_spec = pl.BlockSpec((D, D), lambda i: (0, 0))
_vec_spec = pl.BlockSpec((1, D), lambda i: (0, 0))

_HI = lax.Precision.HIGHEST


def _k1_body(x_ref, w1_ref, d0_ref, d1_ref, xw_ref, y_ref, dis_ref):
    dis = lax.rsqrt(d0_ref[...] + d1_ref[...])
    dis_ref[...] = dis
    xw = lax.dot_general(x_ref[...], w1_ref[...], (((1,), (0,)), ((), ())),
                         precision=_HI)
    xw_ref[...] = xw
    y_ref[...] = dis * xw


def _k3_body(p0_ref, p1_ref, xw1_ref, dis_ref, b1_ref, w2_ref,
             xw2_ref, y2_ref):
    dis = dis_ref[...]
    h = dis * (p0_ref[...] + p1_ref[...]) + dis * dis * xw1_ref[...]
    h = jnp.maximum(h + b1_ref[...], 0.0)
    xw2 = lax.dot_general(h, w2_ref[...], (((1,), (0,)), ((), ())),
                          precision=_HI)
    xw2_ref[...] = xw2
    y2_ref[...] = dis * xw2


def _k5_body(q0_ref, q1_ref, xw2_ref, dis_ref, b2_ref, out_ref):
    dis = dis_ref[...]
    out_ref[...] = (dis * (q0_ref[...] + q1_ref[...])
                    + dis * dis * xw2_ref[...] + b2_ref[...])


_deg_kernel = _make_deg_kernel()
_msg_kernel = _make_msg_kernel()


@jax.jit
def kernel(x, edge_index, edge_weight, W1, b1, W2, b2):
    src3 = jnp.pad(edge_index[0], (0, E_PAD - E)).reshape(NW, MSG_CHUNKS, CH)
    dst3 = jnp.pad(edge_index[1], (0, E_PAD - E)).reshape(NW, MSG_CHUNKS, CH)
    ew3 = jnp.pad(edge_weight, (0, E_PAD - E)).reshape(NW, MSG_CHUNKS, CH)
    xp = jnp.pad(x, ((0, NPAD - N), (0, 0)))

    d0, d1 = _deg_kernel(dst3, ew3)
    d0 = d0.reshape(NPAD, 1)
    d1 = d1.reshape(NPAD, 1)

    xw1, y1, dis = pl.pallas_call(
        _k1_body,
        grid=(NB,),
        in_specs=[_row_spec, _mat_spec, _col_spec, _col_spec],
        out_specs=[_row_spec, _row_spec, _col_spec],
        out_shape=[jax.ShapeDtypeStruct((NPAD, D), jnp.float32),
                   jax.ShapeDtypeStruct((NPAD, D), jnp.float32),
                   jax.ShapeDtypeStruct((NPAD, 1), jnp.float32)],
    )(xp, W1, d0, d1)

    p0, p1 = _msg_kernel(src3, dst3, ew3, y1)

    xw2, y2 = pl.pallas_call(
        _k3_body,
        grid=(NB,),
        in_specs=[_row_spec, _row_spec, _row_spec, _col_spec, _vec_spec,
                  _mat_spec],
        out_specs=[_row_spec, _row_spec],
        out_shape=[jax.ShapeDtypeStruct((NPAD, D), jnp.float32)] * 2,
    )(p0, p1, xw1, dis, b1.reshape(1, D), W2)

    q0, q1 = _msg_kernel(src3, dst3, ew3, y2)

    out = pl.pallas_call(
        _k5_body,
        grid=(NB,),
        in_specs=[_row_spec, _row_spec, _row_spec, _col_spec, _vec_spec],
        out_specs=_row_spec,
        out_shape=jax.ShapeDtypeStruct((NPAD, D), jnp.float32),
    )(q0, q1, xw2, dis, b2.reshape(1, D))
    return out[:N]
